# baseline (speedup 1.0000x reference)
import jax
import jax.numpy as jnp
from jax.experimental import pallas as pl
from jax.experimental.pallas import tpu as pltpu

_PI = 3.141592653589793
_CHUNK = 12544

_S = (9.999791148949e-01, -1.666240153832e-01, 8.308849931241e-03,
      -1.926316995274e-04, 2.147049615597e-06)
_C = (9.999992107412e-01, -4.999942131496e-01, 4.165977758565e-02,
      -1.385878920428e-03, 2.420293205105e-05, -2.197292187089e-07)


def _sincos(v):
    t = v * v
    s = (((_S[4] * t + _S[3]) * t + _S[2]) * t + _S[1]) * t + _S[0]
    s = s * v
    c = ((((_C[5] * t + _C[4]) * t + _C[3]) * t + _C[2]) * t + _C[1]) * t + _C[0]
    return s, c


def _chunks(n):
    out = []
    base = 0
    while base < n:
        w = min(_CHUNK, n - base)
        out.append((base, w))
        base += w
    return out


def _score_kernel(trip_ref, entT_ref, relT_ref, ent_hbm, rel_hbm, out_ref,
                  hg_ref, rg_ref, sem_h, sem_r):
    b_sz = out_ref.shape[0]
    n = out_ref.shape[1]

    copies = []
    for b in range(b_sz):
        h = trip_ref[b, 0]
        r = trip_ref[b, 1]
        ch = pltpu.make_async_copy(
            ent_hbm.at[pl.ds(h, 1), :], hg_ref.at[pl.ds(b, 1), :],
            sem_h.at[b])
        cr = pltpu.make_async_copy(
            rel_hbm.at[pl.ds(r, 1), :], rg_ref.at[pl.ds(b, 1), :],
            sem_r.at[b])
        ch.start()
        cr.start()
        copies.append((ch, cr))

    me = jnp.float32(0)
    for base, w in _chunks(n):
        me = jnp.maximum(me, jnp.max(jnp.abs(entT_ref[:, pl.ds(base, w)])))
    mr = jnp.max(jnp.abs(relT_ref[...]))
    ke = _PI / me
    kr = _PI / mr

    for ch, cr in copies:
        ch.wait()
        cr.wait()
    u = hg_ref[...] * ke + rg_ref[...] * kr
    cu = jnp.cos(u)
    su = jnp.sin(u)

    dn = (((1,), (0,)), ((), ()))
    for base, w in _chunks(n):
        v = entT_ref[:, pl.ds(base, w)] * ke
        s, c = _sincos(v)
        out_ref[:, pl.ds(base, w)] = (
            jax.lax.dot_general(cu, s, dn, preferred_element_type=jnp.float32)
            - jax.lax.dot_general(su, c, dn,
                                  preferred_element_type=jnp.float32))


def kernel(triples, ent_emb, rel_emb):
    batch = triples.shape[0]
    num_ent, dim = ent_emb.shape

    entT = ent_emb.T
    relT = rel_emb.T
    trip = triples.astype(jnp.int32)

    return pl.pallas_call(
        _score_kernel,
        in_specs=[
            pl.BlockSpec(memory_space=pltpu.MemorySpace.SMEM),
            pl.BlockSpec(memory_space=pltpu.MemorySpace.VMEM),
            pl.BlockSpec(memory_space=pltpu.MemorySpace.VMEM),
            pl.BlockSpec(memory_space=pltpu.MemorySpace.HBM),
            pl.BlockSpec(memory_space=pltpu.MemorySpace.HBM),
        ],
        out_specs=pl.BlockSpec(memory_space=pltpu.MemorySpace.VMEM),
        out_shape=jax.ShapeDtypeStruct((batch, num_ent), jnp.float32),
        scratch_shapes=[
            pltpu.VMEM((batch, dim), jnp.float32),
            pltpu.VMEM((batch, dim), jnp.float32),
            pltpu.SemaphoreType.DMA((batch,)),
            pltpu.SemaphoreType.DMA((batch,)),
        ],
    )(trip, entT, relT, ent_emb, rel_emb)
